# baseline (speedup 1.0000x reference)
import functools

import jax
import jax.numpy as jnp
from jax import lax
from jax.experimental import pallas as pl
from jax.experimental.pallas import tpu as pltpu
from jax.experimental.pallas import tpu_sc as plsc

_TC_R = 256
_W = 8


def _sc_gather(table, index_h):
    N, C = table.shape
    BH = index_h.shape[0]
    info = plsc.get_sparse_core_info()
    nw = info.num_cores * info.num_subcores
    L = info.num_lanes
    b_per_w = BH // nw
    nwaves = b_per_w // _W
    mesh = plsc.VectorSubcoreMesh(core_axis_name="c", subcore_axis_name="s")

    @functools.partial(
        pl.kernel,
        mesh=mesh,
        out_type=jax.ShapeDtypeStruct((BH, C), jnp.float32),
        scratch_types=[
            pltpu.VMEM((b_per_w,), jnp.int32),
            pltpu.VMEM((b_per_w, C), jnp.float32),
            pltpu.SemaphoreType.DMA,
        ],
        compiler_params=pltpu.CompilerParams(needs_layout_passes=False),
    )
    def gather(table_hbm, idx_hbm, out_hbm, idx_v, rows_v, sem):
        cid = lax.axis_index("c")
        sid = lax.axis_index("s")
        wid = sid * info.num_cores + cid
        base = wid * b_per_w
        pltpu.sync_copy(idx_hbm.at[pl.ds(base, b_per_w)], idx_v)
        lanes = lax.iota(jnp.int32, L)

        def issue_wave(w):
            def issue_one(j, carry):
                r = w * _W + j
                chunk = idx_v[pl.ds((r // L) * L, L)]
                row = jnp.sum(jnp.where(lanes == lax.rem(r, L), chunk, 0))
                pltpu.make_async_copy(
                    table_hbm.at[pl.ds(row, 1), :],
                    rows_v.at[pl.ds(r, 1), :],
                    sem,
                ).start()
                return carry

            lax.fori_loop(w * 0, _W, issue_one, 0)

        def drain_wave(w):
            pltpu.make_async_copy(
                table_hbm.at[pl.ds(0, _W), :],
                rows_v.at[pl.ds(w * _W, _W), :],
                sem,
            ).wait()

        issue_wave(0)
        for w in range(1, nwaves):
            issue_wave(w)
            drain_wave(w - 1)
        drain_wave(nwaves - 1)
        pltpu.sync_copy(rows_v, out_hbm.at[pl.ds(base, b_per_w)])

    return gather(table, index_h)


def _tc_body(x_ref, g_ref, loss_ref, acc_ref):
    i = pl.program_id(0)

    x = x_ref[...]
    g = g_ref[...]
    m = jnp.max(x, axis=1, keepdims=True)
    lse = m + jnp.log(jnp.sum(jnp.exp(x - m), axis=1, keepdims=True))
    s1 = jnp.sum(g, axis=1, keepdims=True)
    d = jnp.sum(x * g, axis=1, keepdims=True)
    loss = lse * s1 - d
    loss_ref[...] = loss

    @pl.when(i == 0)
    def _():
        acc_ref[...] = jnp.zeros_like(acc_ref)

    acc_ref[...] += jnp.sum(loss).reshape(1, 1)


def _tc_half(outputs, gathered, half):
    B, C = outputs.shape
    BH = gathered.shape[0]
    G = BH // _TC_R
    return pl.pallas_call(
        _tc_body,
        grid=(G,),
        in_specs=[
            pl.BlockSpec((_TC_R, C), lambda i: (i + half * G, 0)),
            pl.BlockSpec((_TC_R, C), lambda i: (i, 0)),
        ],
        out_specs=[
            pl.BlockSpec((_TC_R, 1), lambda i: (i, 0)),
            pl.BlockSpec((1, 1), lambda i: (0, 0)),
        ],
        out_shape=[
            jax.ShapeDtypeStruct((BH, 1), jnp.float32),
            jax.ShapeDtypeStruct((1, 1), jnp.float32),
        ],
    )(outputs, gathered)


def kernel(outputs, index, confidence):
    B, C = outputs.shape
    BH = B // 2
    ga = _sc_gather(confidence, index[:BH])
    gb = _sc_gather(confidence, index[BH:])
    loss_a, sum_a = _tc_half(outputs, ga, 0)
    loss_b, sum_b = _tc_half(outputs, gb, 1)
    avg = (sum_a[0, 0] + sum_b[0, 0]) / B
    loss_vec = jnp.concatenate([loss_a, loss_b], axis=0).reshape(B)
    return (avg, loss_vec)
